# baseline (speedup 1.0000x reference)
import functools

import jax
import jax.numpy as jnp
from jax import lax
from jax.experimental import pallas as pl
from jax.experimental.pallas import tpu as pltpu
from jax.experimental.pallas import tpu_sc as plsc

ALPHA = 20
KMER = 5
MOTIF_EFFECT = 2.0
NC, NS, LANES = 2, 16, 16
NW = NC * NS
GROUPS = 8
GROW = 64


def _wid_base(RPW):
    cid = lax.axis_index("c")
    sid = lax.axis_index("s")
    return (sid * NC + cid) * RPW


def _idx_body(B, Lseq, cand_hbm, idxh_hbm, slab, idx_v, ssem, osem):
    W = Lseq - KMER + 1
    RPW = B // NW
    GCH = GROW // LANES
    GSZ = W * GROW
    POW_TOP = ALPHA ** (KMER - 1)
    base = _wid_base(RPW)

    def slab_dma(g):
        return pltpu.make_async_copy(
            cand_hbm.at[pl.ds(base + g * GROW, GROW), :],
            slab.at[pl.ds(g * GROW, GROW), :], ssem.at[g])

    def out_dma(g):
        return pltpu.make_async_copy(
            idx_v.at[pl.ds(g * GSZ, GSZ)],
            idxh_hbm.at[pl.ds(base * W + g * GSZ, GSZ)], osem.at[g])

    for g in range(GROUPS):
        slab_dma(g).start()

    def compute_idx(g):
        def idx_chunk(c, carry):
            rows = (g * GROW + c * LANES) + lax.iota(jnp.int32, LANES)
            cols = {}

            def col(j):
                if j not in cols:
                    cols[j] = plsc.load_gather(
                        slab, [rows, jnp.full((LANES,), j, jnp.int32)])
                return cols[j]

            h = col(0)
            for j in range(1, KMER):
                h = h * ALPHA + col(j)
            idx_v[pl.ds(g * GSZ + c * LANES, LANES)] = h
            for w in range(1, W):
                h = (h - col(w - 1) * POW_TOP) * ALPHA + col(w + KMER - 1)
                idx_v[pl.ds(g * GSZ + w * GROW + c * LANES, LANES)] = h
            return carry

        lax.fori_loop(0, GCH, idx_chunk, None)

    for g in range(GROUPS):
        slab_dma(g).wait()
        compute_idx(g)
        out_dma(g).start()
    for g in range(GROUPS):
        out_dma(g).wait()


def _gather_body(B, Lseq, P, tab_hbm, idxh_hbm, ncb_hbm, out_hbm,
                 ncb_v, encb_v, idx_v, val_v, out_v, isem, gsem):
    W = Lseq - KMER + 1
    RPW = B // NW
    GCH = GROW // LANES
    GSZ = W * GROW
    base = _wid_base(RPW)

    def idx_dma(g):
        return pltpu.make_async_copy(
            idxh_hbm.at[pl.ds(base * W + g * GSZ, GSZ)],
            idx_v.at[pl.ds(g * GSZ, GSZ)], isem.at[g])

    def gather_dma(g):
        return pltpu.make_async_copy(
            tab_hbm.at[idx_v.at[pl.ds(g * GSZ, GSZ)]],
            val_v.at[pl.ds(g * GSZ, GSZ)], gsem.at[g])

    for g in range(GROUPS):
        idx_dma(g).start()
    pltpu.sync_copy(ncb_hbm, ncb_v)
    for p in range(P):
        encb_v[p, :] = jnp.exp(-ncb_v[p, :])

    for g in range(GROUPS):
        idx_dma(g).wait()
        gather_dma(g).start()

    inv_p = 1.0 / P

    def reduce_group(g):
        def red_chunk(c, carry):
            cnt = val_v[pl.ds(g * GSZ + c * LANES, LANES)]
            for w in range(1, W):
                cnt = cnt + val_v[pl.ds(g * GSZ + w * GROW + c * LANES, LANES)]
            contrib = jnp.minimum(cnt, 1.0) * MOTIF_EFFECT
            t = jnp.exp(-contrib)
            s = jnp.zeros((LANES,), jnp.float32)
            for p in range(P):
                s = s + 1.0 / (1.0 + t * encb_v[p, :])
            out_v[pl.ds(g * GROW + c * LANES, LANES)] = s * inv_p
            return carry

        lax.fori_loop(0, GCH, red_chunk, None)

    for g in range(GROUPS):
        gather_dma(g).wait()
        reduce_group(g)

    pltpu.sync_copy(out_v, out_hbm.at[pl.ds(base, RPW)])


@jax.jit
def kernel(candidates, table, natural_contribs):
    B, Lseq = candidates.shape
    P = natural_contribs.shape[0]
    W = Lseq - KMER + 1
    RPW = B // NW

    cand = candidates.astype(jnp.int32)
    tab = table.reshape(-1).astype(jnp.float32)
    ncb = jnp.broadcast_to(
        natural_contribs.astype(jnp.float32)[:, None], (P, LANES))

    mesh = plsc.VectorSubcoreMesh(
        core_axis_name="c", subcore_axis_name="s",
        num_cores=NC, num_subcores=NS)
    params = pltpu.CompilerParams(needs_layout_passes=False)

    idx_run = pl.kernel(
        functools.partial(_idx_body, B, Lseq),
        out_type=jax.ShapeDtypeStruct((B * W,), jnp.int32),
        mesh=mesh,
        compiler_params=params,
        scratch_types=[
            pltpu.VMEM((RPW, Lseq), jnp.int32),
            pltpu.VMEM((GROUPS * W * GROW,), jnp.int32),
            pltpu.SemaphoreType.DMA((GROUPS,)),
            pltpu.SemaphoreType.DMA((GROUPS,)),
        ],
    )
    idx_hbm = idx_run(cand)

    gather_run = pl.kernel(
        functools.partial(_gather_body, B, Lseq, P),
        out_type=jax.ShapeDtypeStruct((B,), jnp.float32),
        mesh=mesh,
        compiler_params=params,
        scratch_types=[
            pltpu.VMEM((P, LANES), jnp.float32),
            pltpu.VMEM((P, LANES), jnp.float32),
            pltpu.VMEM((GROUPS * W * GROW,), jnp.int32),
            pltpu.VMEM((GROUPS * W * GROW,), jnp.float32),
            pltpu.VMEM((RPW,), jnp.float32),
            pltpu.SemaphoreType.DMA((GROUPS,)),
            pltpu.SemaphoreType.DMA((GROUPS,)),
        ],
    )
    return gather_run(tab, idx_hbm, ncb)


# speedup vs baseline: 1.4268x; 1.4268x over previous
import functools

import jax
import jax.numpy as jnp
from jax import lax
from jax.experimental import pallas as pl
from jax.experimental.pallas import tpu as pltpu
from jax.experimental.pallas import tpu_sc as plsc

ALPHA = 20
KMER = 5
MOTIF_EFFECT = 2.0
NC, NS, LANES = 2, 16, 16
NW = NC * NS
GROUPS = 8
GROW = 64


def _body(B, Lseq, P, cand_hbm, table_hbm, nc_hbm, out_hbm,
          slab, nc_v, encb_v, idx_v, val_v, out_v, sem, ssem):
    W = Lseq - KMER + 1
    RPW = B // NW
    GCH = GROW // LANES
    GSZ = W * GROW
    POW_TOP = ALPHA ** (KMER - 1)

    cid = lax.axis_index("c")
    sid = lax.axis_index("s")
    wid = sid * NC + cid
    base = wid * RPW

    def slab_dma(g):
        return pltpu.make_async_copy(
            cand_hbm.at[pl.ds(base + g * GROW, GROW), :],
            slab.at[pl.ds(g * GROW, GROW), :], ssem.at[g])

    for g in range(GROUPS):
        slab_dma(g).start()
    pltpu.sync_copy(nc_hbm, nc_v)
    for p in range(P):
        ncp = plsc.load_gather(nc_v, [jnp.full((LANES,), p, jnp.int32)])
        encb_v[p, :] = jnp.exp(-ncp)

    zeros16 = jnp.zeros((LANES,), jnp.int32)

    def compute_idx(g):
        def idx_chunk(c, carry):
            rows = (g * GROW + c * LANES) + lax.iota(jnp.int32, LANES)
            cols = {}

            def col(j):
                if j not in cols:
                    cols[j] = plsc.load_gather(
                        slab, [rows, jnp.full((LANES,), j, jnp.int32)])
                return cols[j]

            h = col(0)
            for j in range(1, KMER):
                h = h * ALPHA + col(j)
            idx_v[pl.ds(g * GSZ + c * LANES, LANES)] = h
            for w in range(1, W):
                h = (h - col(w - 1) * POW_TOP) * ALPHA + col(w + KMER - 1)
                idx_v[pl.ds(g * GSZ + w * GROW + c * LANES, LANES)] = h
            return carry

        lax.fori_loop(0, GCH, idx_chunk, None)

    def gather_dma(g):
        return pltpu.make_async_copy(
            table_hbm.at[idx_v.at[pl.ds(g * GSZ, GSZ)]],
            val_v.at[pl.ds(g * GSZ, GSZ)], sem.at[g])

    inv_p = 1.0 / P

    def reduce_group(g):
        def red_chunk(c, carry):
            cnt = val_v[pl.ds(g * GSZ + c * LANES, LANES)]
            for w in range(1, W):
                cnt = cnt + val_v[pl.ds(g * GSZ + w * GROW + c * LANES, LANES)]
            contrib = jnp.minimum(cnt, 1.0) * MOTIF_EFFECT
            t = jnp.exp(-contrib)
            s = jnp.zeros((LANES,), jnp.float32)
            for p in range(P):
                s = s + 1.0 / (1.0 + t * encb_v[p, :])
            out_v[pl.ds(g * GROW + c * LANES, LANES)] = s * inv_p
            return carry

        lax.fori_loop(0, GCH, red_chunk, None)

    slab_dma(0).wait()
    compute_idx(0)
    gather_dma(0).start()
    for g in range(1, GROUPS):
        slab_dma(g).wait()
        compute_idx(g)
        gather_dma(g).start()
    for g in range(GROUPS):
        gather_dma(g).wait()
        reduce_group(g)

    pltpu.sync_copy(out_v, out_hbm.at[pl.ds(base, RPW)])


@jax.jit
def kernel(candidates, table, natural_contribs):
    B, Lseq = candidates.shape
    P = natural_contribs.shape[0]
    W = Lseq - KMER + 1
    RPW = B // NW

    cand = candidates.astype(jnp.int32)
    tab = table.reshape(-1).astype(jnp.float32)

    mesh = plsc.VectorSubcoreMesh(
        core_axis_name="c", subcore_axis_name="s",
        num_cores=NC, num_subcores=NS)

    run = pl.kernel(
        functools.partial(_body, B, Lseq, P),
        out_type=jax.ShapeDtypeStruct((B,), jnp.float32),
        mesh=mesh,
        compiler_params=pltpu.CompilerParams(needs_layout_passes=False),
        scratch_types=[
            pltpu.VMEM((RPW, Lseq), jnp.int32),
            pltpu.VMEM((P,), jnp.float32),
            pltpu.VMEM((P, LANES), jnp.float32),
            pltpu.VMEM((GROUPS * W * GROW,), jnp.int32),
            pltpu.VMEM((GROUPS * W * GROW,), jnp.float32),
            pltpu.VMEM((RPW,), jnp.float32),
            pltpu.SemaphoreType.DMA((GROUPS,)),
            pltpu.SemaphoreType.DMA((GROUPS,)),
        ],
    )
    return run(cand, tab, natural_contribs.astype(jnp.float32))
